# Initial kernel scaffold; baseline (speedup 1.0000x reference)
import jax
import jax.numpy as jnp
from jax.experimental import pallas as pl


def kernel(x, edge_index, edge_attr, batch, W1, b1, p1, W2, b2, p2, lw1, lb1, lw2, lb2, lw3, lb3):
    raise NotImplementedError("write your pallas kernel here")


# speedup vs baseline: 1.0000x; 1.0000x over previous
import functools

import jax
import jax.numpy as jnp
from jax.experimental import pallas as pl
from jax.experimental.pallas import tpu as pltpu


def _gcn(x, src, dst, ew, W, b, n):
    loop = jnp.arange(n)
    s = jnp.concatenate([src, loop])
    d = jnp.concatenate([dst, loop])
    w = jnp.concatenate([ew, jnp.ones((n,), x.dtype)])
    deg = jnp.zeros((n,), x.dtype).at[d].add(w)
    dis = jnp.where(deg > 0, 1.0 / jnp.sqrt(deg), 0.0)
    norm = dis[s] * w * dis[d]
    xw = x @ W
    return jnp.zeros((n, W.shape[1]), x.dtype).at[d].add(norm[:, None] * xw[s]) + b


def _mlp_kernel(xin_ref, lw1_ref, lb1_ref, lw2_ref, lb2_ref, lw3_ref, lb3_ref, out_ref):
    h = xin_ref[...] @ lw1_ref[...] + lb1_ref[...]
    h = h @ lw2_ref[...] + lb2_ref[...]
    out_ref[...] = h @ lw3_ref[...] + lb3_ref[...]


def _mlp(xin, lw1, lb1, lw2, lb2, lw3, lb3):
    g = xin.shape[0]
    return pl.pallas_call(
        _mlp_kernel,
        out_shape=jax.ShapeDtypeStruct((g, lw3.shape[1]), xin.dtype),
    )(xin, lw1, lb1[None, :], lw2, lb2[None, :], lw3, lb3[None, :])


def kernel(x, edge_index, edge_attr, batch, W1, b1, p1, W2, b2, p2, lw1, lb1, lw2, lb2, lw3, lb3):
    ratio = 0.8
    n = x.shape[0]
    g = 16
    src, dst = edge_index[0], edge_index[1]
    h = jax.nn.relu(_gcn(x, src, dst, edge_attr, W1, b1, n))
    s1 = jnp.tanh((h @ p1) / jnp.linalg.norm(p1))
    bi = batch.astype(jnp.int32)
    idx = jnp.arange(n, dtype=jnp.int32)
    bs, _, perm1 = jax.lax.sort((bi, -s1, idx), num_keys=2, is_stable=True)
    cnt = jax.ops.segment_sum(jnp.ones((n,), jnp.int32), bi, num_segments=g)
    start = jnp.concatenate([jnp.zeros((1,), jnp.int32), jnp.cumsum(cnt)[:-1]])
    rank1 = idx - start[bs]
    k1 = (4 * cnt + 4) // 5
    sel1 = rank1 < k1[bs]
    m1 = sel1.astype(x.dtype)[:, None]
    h1 = h[perm1] * s1[perm1][:, None] * m1
    pos = jnp.zeros((n,), jnp.int32).at[perm1].set(jnp.where(sel1, idx, -1))
    ms, md = pos[src], pos[dst]
    keep = (ms >= 0) & (md >= 0)
    src1 = jnp.where(keep, ms, 0)
    dst1 = jnp.where(keep, md, 0)
    ea1 = jnp.where(keep, edge_attr, jnp.zeros_like(edge_attr))
    x1 = jax.ops.segment_sum(h1, bs, num_segments=g) / jnp.maximum(
        jax.ops.segment_sum(m1, bs, num_segments=g), 1.0)
    h2 = jax.nn.relu(_gcn(h1, src1, dst1, ea1, W2, b2, n))
    s2 = jnp.tanh((h2 @ p2) / jnp.linalg.norm(p2))
    nsel = 1 - sel1.astype(jnp.int32)
    c1 = jax.ops.segment_sum(sel1.astype(jnp.int32), bs, num_segments=g)
    _, _, _, perm2 = jax.lax.sort((bs, nsel, -s2, idx), num_keys=3, is_stable=True)
    rank2 = idx - start[bs]
    k2 = (4 * c1 + 4) // 5
    sel2 = rank2 < k2[bs]
    m2 = sel2.astype(x.dtype)[:, None]
    h2p = h2[perm2] * s2[perm2][:, None] * m2
    x2 = jax.ops.segment_sum(h2p, bs, num_segments=g) / jnp.maximum(
        jax.ops.segment_sum(m2, bs, num_segments=g), 1.0)
    return _mlp(x1 + x2, lw1, lb1, lw2, lb2, lw3, lb3)


# speedup vs baseline: 15.6789x; 15.6788x over previous
import functools

import jax
import jax.numpy as jnp
from jax import lax
from jax.experimental import pallas as pl
from jax.experimental.pallas import tpu as pltpu
from jax.experimental.pallas import tpu_sc as plsc

N = 10000
E = 320000
D = 128
G = 16
NP = 10240
EP = 327680
ER = EP // 128
NC, NS = 2, 16
NPT = NP // NS

F32 = jnp.float32
I32 = jnp.int32


_DEG_ROWS = ER // NS
_DEG_WIN = 4
_MSG_ROWS = ER // (NC * NS)

def _nrsqrt(d):
    bits = plsc.bitcast(d, I32)
    y = plsc.bitcast(jnp.int32(0x5F3759DF) - (bits >> 1), F32)
    for _ in range(3):
        y = y * (1.5 - 0.5 * d * y * y)
    return y


@functools.cache
def _build_sc_conv():
    mesh = plsc.VectorSubcoreMesh(
        core_axis_name="c", subcore_axis_name="s",
        num_cores=NC, num_subcores=NS)
    return pl.kernel(
        _sc_conv_body,
        (jax.ShapeDtypeStruct((NC, NP, D), F32),
         jax.ShapeDtypeStruct((NC, NP), F32)),
        mesh=mesh,
        compiler_params=pltpu.CompilerParams(
            use_tc_tiling_on_sc=False, needs_layout_passes=False),
        scratch_types=dict(
            nfd_t=pltpu.VMEM((NP,), F32),
            dsrc=pltpu.VMEM((_DEG_WIN, 128), I32),
            ddst=pltpu.VMEM((_DEG_WIN, 128), I32),
            dea=pltpu.VMEM((_DEG_WIN, 128), F32),
            dval=pltpu.VMEM((_DEG_WIN, 128), F32),
            msrc=pltpu.VMEM((1, 128), I32),
            mdst=pltpu.VMEM((1, 128), I32),
            mea=pltpu.VMEM((1, 128), F32),
            scl=pltpu.VMEM((144,), F32),
            rows=pltpu.VMEM((128, D), F32),
            degbuf=pltpu.VMEM((NPT,), F32),
            sem=pltpu.SemaphoreType.DMA,
            acc_sh=pltpu.VMEM_SHARED((NP, D), F32),
            deg_sh=pltpu.VMEM_SHARED((NP,), F32),
            dsx_sh=pltpu.VMEM_SHARED((NP,), F32),
        ),
    )


def _sc_conv(srcp, dstp, eap, nf, xw):
    return _build_sc_conv()(srcp, dstp, eap, nf, xw)


def _sc_conv_body(src_h, dst_h, ea_h, nf_h, xw_h, acc_o, raw_o, *,
             nfd_t, dsrc, ddst, dea, dval, msrc, mdst, mea, scl,
             rows, degbuf, sem, acc_sh, deg_sh, dsx_sh):
    c = lax.axis_index("c")
    s = lax.axis_index("s")
    w = c * NS + s
    zeros16 = jnp.zeros((16,), F32)

    pltpu.sync_copy(nf_h, nfd_t)

    def _zrow(i, t):
        for k in range(D // 16):
            rows[i, pl.ds(k * 16, 16)] = zeros16
        return t
    lax.fori_loop(0, 128, _zrow, 0)

    def _zdeg(i, t):
        degbuf[pl.ds(i * 16, 16)] = zeros16
        return t
    lax.fori_loop(0, NPT // 16, _zdeg, 0)

    for k in range(NPT // 128):
        pltpu.sync_copy(rows, acc_sh.at[pl.ds(s * NPT + k * 128, 128)])
    pltpu.sync_copy(degbuf, deg_sh.at[pl.ds(s * NPT, NPT)])
    plsc.subcore_barrier()

    def _deg_win(win, t):
        r0 = s * _DEG_ROWS + win * _DEG_WIN
        pltpu.sync_copy(src_h.at[pl.ds(r0, _DEG_WIN)], dsrc)
        pltpu.sync_copy(dst_h.at[pl.ds(r0, _DEG_WIN)], ddst)
        pltpu.sync_copy(ea_h.at[pl.ds(r0, _DEG_WIN)], dea)

        def _crow(j, u):
            for k in range(8):
                sidx = dsrc[j, pl.ds(k * 16, 16)]
                nfv = plsc.load_gather(nfd_t, [sidx])
                dval[j, pl.ds(k * 16, 16)] = nfv * dea[j, pl.ds(k * 16, 16)]
            return u
        lax.fori_loop(0, _DEG_WIN, _crow, 0)

        def _srow(j, u):
            pltpu.sync_copy(dval.at[j], deg_sh.at[ddst.at[j]], add=True)
            return u
        lax.fori_loop(0, _DEG_WIN, _srow, 0)
        return t
    lax.fori_loop(0, _DEG_ROWS // _DEG_WIN, _deg_win, 0)
    plsc.subcore_barrier()

    base = s * NPT
    pltpu.sync_copy(deg_sh.at[pl.ds(base, NPT)], degbuf)
    pltpu.sync_copy(degbuf, raw_o.at[c, pl.ds(base, NPT)])

    def _dis(i, t):
        raw = degbuf[pl.ds(i * 16, 16)]
        nfv = nfd_t[pl.ds(base + i * 16, 16)]
        d = 1.0 + nfv * raw
        degbuf[pl.ds(i * 16, 16)] = _nrsqrt(d) * nfv
        return t
    lax.fori_loop(0, NPT // 16, _dis, 0)
    pltpu.sync_copy(degbuf, dsx_sh.at[pl.ds(base, NPT)])
    plsc.subcore_barrier()
    pltpu.sync_copy(dsx_sh, nfd_t)

    def _msg_win(win, t):
        r0 = w * _MSG_ROWS + win
        pltpu.sync_copy(src_h.at[r0], msrc.at[0])
        pltpu.sync_copy(dst_h.at[r0], mdst.at[0])
        pltpu.sync_copy(ea_h.at[r0], mea.at[0])
        pltpu.async_copy(xw_h.at[msrc.at[0]], rows, sem).wait()
        for k in range(8):
            sidx = msrc[0, pl.ds(k * 16, 16)]
            g16 = plsc.load_gather(nfd_t, [sidx])
            scl[pl.ds(k * 16, 16)] = g16 * mea[0, pl.ds(k * 16, 16)]

        def _erow(j, u):
            sc = scl[pl.ds(j, 16)][0]
            for m in range(D // 16):
                rows[j, pl.ds(m * 16, 16)] = rows[j, pl.ds(m * 16, 16)] * sc
            return u
        lax.fori_loop(0, 128, _erow, 0)
        pltpu.sync_copy(rows, acc_sh.at[mdst.at[0]], add=True)
        return t
    lax.fori_loop(0, _MSG_ROWS, _msg_win, 0)
    plsc.subcore_barrier()

    pltpu.sync_copy(acc_sh.at[pl.ds(base, NPT)],
                    acc_o.at[c, pl.ds(base, NPT)])


_RB = 1024


def _mm_body(x_ref, w_ref, o_ref):
    o_ref[...] = jnp.dot(x_ref[...], w_ref[...],
                         preferred_element_type=F32)


def _mm(x, w):
    return pl.pallas_call(
        _mm_body,
        grid=(NP // _RB,),
        in_specs=[pl.BlockSpec((_RB, D), lambda i: (i, 0)),
                  pl.BlockSpec((D, D), lambda i: (0, 0))],
        out_specs=pl.BlockSpec((_RB, D), lambda i: (i, 0)),
        out_shape=jax.ShapeDtypeStruct((NP, D), F32),
    )(x, w)


def _mm2_body(h_ref, s_ref, sel_ref, w_ref, o_ref):
    hm = h_ref[...] * (s_ref[...] * sel_ref[...])
    o_ref[...] = jnp.dot(hm, w_ref[...], preferred_element_type=F32)


def _mm_masked(h, s, sel, w):
    return pl.pallas_call(
        _mm2_body,
        grid=(NP // _RB,),
        in_specs=[pl.BlockSpec((_RB, D), lambda i: (i, 0)),
                  pl.BlockSpec((_RB, 1), lambda i: (i, 0)),
                  pl.BlockSpec((_RB, 1), lambda i: (i, 0)),
                  pl.BlockSpec((D, D), lambda i: (0, 0))],
        out_specs=pl.BlockSpec((_RB, D), lambda i: (i, 0)),
        out_shape=jax.ShapeDtypeStruct((NP, D), F32),
    )(h, s, sel, w)


def _mid_body(acc0_ref, acc1_ref, raw_ref, nf_ref, xw_ref, b_ref, p_ref,
              h_ref, s_ref):
    nf = nf_ref[...]
    deg = 1.0 + nf * raw_ref[...]
    dsx = lax.rsqrt(deg) * nf
    h = (acc0_ref[...] + acc1_ref[...]) * dsx \
        + xw_ref[...] * (1.0 / deg) + b_ref[...]
    h = jnp.maximum(h, 0.0)
    h_ref[...] = h
    p = p_ref[...]
    pn = lax.rsqrt(jnp.sum(p * p))
    s_ref[...] = jnp.tanh(jnp.dot(h, p, preferred_element_type=F32) * pn)


def _mid(acc0, acc1, raw, nf, xw, b, p):
    return pl.pallas_call(
        _mid_body,
        grid=(NP // _RB,),
        in_specs=[pl.BlockSpec((_RB, D), lambda i: (i, 0)),
                  pl.BlockSpec((_RB, D), lambda i: (i, 0)),
                  pl.BlockSpec((_RB, 1), lambda i: (i, 0)),
                  pl.BlockSpec((_RB, 1), lambda i: (i, 0)),
                  pl.BlockSpec((_RB, D), lambda i: (i, 0)),
                  pl.BlockSpec((1, D), lambda i: (0, 0)),
                  pl.BlockSpec((D, 1), lambda i: (0, 0))],
        out_specs=[pl.BlockSpec((_RB, D), lambda i: (i, 0)),
                   pl.BlockSpec((_RB, 1), lambda i: (i, 0))],
        out_shape=[jax.ShapeDtypeStruct((NP, D), F32),
                   jax.ShapeDtypeStruct((NP, 1), F32)],
    )(acc0, acc1, raw, nf, xw, b, p)


_IB = 256
_JC = 512


def _rank1_body(jlo_ref, jhi_ref, scol_ref, bcol_ref, srow_ref, brow_ref,
                rank_ref):
    pid = pl.program_id(0)
    si = scol_ref[...]
    bi = bcol_ref[...]
    ii = _IB * pid + lax.broadcasted_iota(I32, (_IB, 1), 0)

    def jbody(cb, acc):
        sj = srow_ref[:, pl.ds(cb * _JC, _JC)]
        bj = brow_ref[:, pl.ds(cb * _JC, _JC)]
        jj = cb * _JC + lax.broadcasted_iota(I32, (1, _JC), 1)
        cmp = (bj == bi) & ((sj > si) | ((sj == si) & (jj < ii)))
        return acc + jnp.sum(cmp.astype(I32), axis=1, keepdims=True)

    rank_ref[...] = lax.fori_loop(jlo_ref[pid], jhi_ref[pid], jbody,
                                  jnp.zeros((_IB, 1), I32))


def _rank1(jlo, jhi, scol, bcol, srow, brow):
    return pl.pallas_call(
        _rank1_body,
        grid=(NP // _IB,),
        in_specs=[pl.BlockSpec(memory_space=pltpu.SMEM),
                  pl.BlockSpec(memory_space=pltpu.SMEM),
                  pl.BlockSpec((_IB, 1), lambda i: (i, 0)),
                  pl.BlockSpec((_IB, 1), lambda i: (i, 0)),
                  pl.BlockSpec((1, NP), lambda i: (0, 0)),
                  pl.BlockSpec((1, NP), lambda i: (0, 0))],
        out_specs=pl.BlockSpec((_IB, 1), lambda i: (i, 0)),
        out_shape=jax.ShapeDtypeStruct((NP, 1), I32),
    )(jlo, jhi, scol, bcol, srow, brow)


def _rank2_body(jlo_ref, jhi_ref, s2c_ref, s1c_ref, bcol_ref,
                s2r_ref, s1r_ref, brow_ref, selr_ref, rank_ref):
    pid = pl.program_id(0)
    s2i = s2c_ref[...]
    s1i = s1c_ref[...]
    bi = bcol_ref[...]
    ii = _IB * pid + lax.broadcasted_iota(I32, (_IB, 1), 0)

    def jbody(cb, acc):
        s2j = s2r_ref[:, pl.ds(cb * _JC, _JC)]
        s1j = s1r_ref[:, pl.ds(cb * _JC, _JC)]
        bj = brow_ref[:, pl.ds(cb * _JC, _JC)]
        selj = selr_ref[:, pl.ds(cb * _JC, _JC)] > 0.5
        jj = cb * _JC + lax.broadcasted_iota(I32, (1, _JC), 1)
        before = (s1j > s1i) | ((s1j == s1i) & (jj < ii))
        cmp = (bj == bi) & selj & ((s2j > s2i) | ((s2j == s2i) & before))
        return acc + jnp.sum(cmp.astype(I32), axis=1, keepdims=True)

    rank_ref[...] = lax.fori_loop(jlo_ref[pid], jhi_ref[pid], jbody,
                                  jnp.zeros((_IB, 1), I32))


def _rank2(jlo, jhi, s2c, s1c, bcol, s2r, s1r, brow, selr):
    return pl.pallas_call(
        _rank2_body,
        grid=(NP // _IB,),
        in_specs=[pl.BlockSpec(memory_space=pltpu.SMEM),
                  pl.BlockSpec(memory_space=pltpu.SMEM),
                  pl.BlockSpec((_IB, 1), lambda i: (i, 0)),
                  pl.BlockSpec((_IB, 1), lambda i: (i, 0)),
                  pl.BlockSpec((_IB, 1), lambda i: (i, 0)),
                  pl.BlockSpec((1, NP), lambda i: (0, 0)),
                  pl.BlockSpec((1, NP), lambda i: (0, 0)),
                  pl.BlockSpec((1, NP), lambda i: (0, 0)),
                  pl.BlockSpec((1, NP), lambda i: (0, 0))],
        out_specs=pl.BlockSpec((_IB, 1), lambda i: (i, 0)),
        out_shape=jax.ShapeDtypeStruct((NP, 1), I32),
    )(jlo, jhi, s2c, s1c, bcol, s2r, s1r, brow, selr)


def _pool_body(bcol_ref, h1_ref, w1_ref, h2_ref, w2_ref, inv1_ref, inv2_ref,
               lw1_ref, lb1_ref, lw2_ref, lb2_ref, lw3_ref, lb3_ref,
               out_ref, a1_s, a2_s):
    pid = pl.program_id(0)

    @pl.when(pid == 0)
    def _():
        a1_s[...] = jnp.zeros_like(a1_s)
        a2_s[...] = jnp.zeros_like(a2_s)

    oh = (bcol_ref[...] == lax.broadcasted_iota(I32, (1, G), 1)).astype(F32)
    hm1 = h1_ref[...] * w1_ref[...]
    hm2 = h2_ref[...] * w2_ref[...]
    dn = (((0,), (0,)), ((), ()))
    a1_s[...] += lax.dot_general(oh, hm1, dn, preferred_element_type=F32)
    a2_s[...] += lax.dot_general(oh, hm2, dn, preferred_element_type=F32)

    @pl.when(pid == NP // _RB - 1)
    def _():
        xx = a1_s[...] * inv1_ref[...] + a2_s[...] * inv2_ref[...]
        o = jnp.dot(xx, lw1_ref[...], preferred_element_type=F32) + lb1_ref[...]
        o = jnp.dot(o, lw2_ref[...], preferred_element_type=F32) + lb2_ref[...]
        o = jnp.dot(o, lw3_ref[...], preferred_element_type=F32) + lb3_ref[...]
        out_ref[...] = o


def _pool_mlp(bcol, h1, w1, h2, w2, inv1, inv2, lw1, lb1, lw2, lb2, lw3, lb3):
    no = lw3.shape[1]
    return pl.pallas_call(
        _pool_body,
        grid=(NP // _RB,),
        in_specs=[pl.BlockSpec((_RB, 1), lambda i: (i, 0)),
                  pl.BlockSpec((_RB, D), lambda i: (i, 0)),
                  pl.BlockSpec((_RB, 1), lambda i: (i, 0)),
                  pl.BlockSpec((_RB, D), lambda i: (i, 0)),
                  pl.BlockSpec((_RB, 1), lambda i: (i, 0)),
                  pl.BlockSpec((G, 1), lambda i: (0, 0)),
                  pl.BlockSpec((G, 1), lambda i: (0, 0)),
                  pl.BlockSpec((D, D), lambda i: (0, 0)),
                  pl.BlockSpec((1, D), lambda i: (0, 0)),
                  pl.BlockSpec((D, 64), lambda i: (0, 0)),
                  pl.BlockSpec((1, 64), lambda i: (0, 0)),
                  pl.BlockSpec((64, no), lambda i: (0, 0)),
                  pl.BlockSpec((1, no), lambda i: (0, 0))],
        out_specs=pl.BlockSpec((G, no), lambda i: (0, 0)),
        out_shape=jax.ShapeDtypeStruct((G, no), F32),
        scratch_shapes=[pltpu.VMEM((G, D), F32), pltpu.VMEM((G, D), F32)],
    )(bcol, h1, w1, h2, w2, inv1, inv2, lw1, lb1, lw2, lb2, lw3, lb3)


def kernel(x, edge_index, edge_attr, batch, W1, b1, p1, W2, b2, p2,
           lw1, lb1, lw2, lb2, lw3, lb3):
    src = edge_index[0].astype(I32)
    dst = edge_index[1].astype(I32)
    pe = EP - E
    padi = (jnp.arange(pe, dtype=I32) * 37) % N
    srcp = jnp.concatenate([src, padi]).reshape(ER, 128)
    dstp = jnp.concatenate([dst, padi]).reshape(ER, 128)
    eap = jnp.concatenate([edge_attr.astype(F32),
                           jnp.zeros((pe,), F32)]).reshape(ER, 128)
    xp = jnp.concatenate([x.astype(F32), jnp.zeros((NP - N, D), F32)])
    batchp = jnp.concatenate(
        [batch.astype(I32), jnp.full((NP - N,), G, I32)])
    bcol = batchp[:, None]
    brow = batchp[None, :]

    edges = jnp.searchsorted(batchp, jnp.arange(G + 1, dtype=I32),
                             side="left").astype(I32)
    cnt = edges[1:] - edges[:-1]
    k1 = (4 * cnt + 4) // 5
    k2 = (4 * k1 + 4) // 5
    k1x = jnp.concatenate([k1, jnp.zeros((1,), I32)])
    k2x = jnp.concatenate([k2, jnp.zeros((1,), I32)])

    bfirst = batchp[0::_IB]
    blast = batchp[_IB - 1::_IB]
    jlo = (jnp.searchsorted(batchp, bfirst, side="left") // _JC).astype(I32)
    jhi = ((jnp.searchsorted(batchp, blast, side="right") + _JC - 1)
           // _JC).astype(I32)

    ones_nf = jnp.ones((NP,), F32)

    xw1 = _mm(xp, W1)
    accp1, rawp1 = _sc_conv(srcp, dstp, eap, ones_nf, xw1)
    h1, s1 = _mid(accp1[0], accp1[1], rawp1[0][:, None], ones_nf[:, None],
                  xw1, b1[None, :], p1[:, None])

    s1row = s1.reshape(1, NP)
    rank1 = _rank1(jlo, jhi, s1, bcol, s1row, brow)
    sel1 = (rank1[:, 0] < k1x[batchp]).astype(F32)
    sel1c = sel1[:, None]

    xw2 = _mm_masked(h1, s1, sel1c, W2)
    accp2, rawp2 = _sc_conv(srcp, dstp, eap, sel1, xw2)
    h2, s2 = _mid(accp2[0], accp2[1], rawp2[0][:, None], sel1c,
                  xw2, b2[None, :], p2[:, None])

    rank2 = _rank2(jlo, jhi, s2, s1, bcol, s2.reshape(1, NP), s1row, brow,
                   sel1.reshape(1, NP))
    sel2 = sel1 * (rank2[:, 0] < k2x[batchp]).astype(F32)

    inv1 = (1.0 / jnp.maximum(k1.astype(F32), 1.0))[:, None]
    inv2 = (1.0 / jnp.maximum(k2.astype(F32), 1.0))[:, None]
    return _pool_mlp(bcol, h1, s1 * sel1c, h2, s2 * sel2[:, None],
                     inv1, inv2, lw1, lb1[None, :], lw2, lb2[None, :],
                     lw3, lb3[None, :])
